# Initial kernel scaffold; baseline (speedup 1.0000x reference)
import jax
import jax.numpy as jnp
from jax.experimental import pallas as pl


def kernel(x, emb_table, pos_table):
    raise NotImplementedError("write your pallas kernel here")


# speedup vs baseline: 1.1450x; 1.1450x over previous
import functools

import jax
import jax.numpy as jnp
from jax import lax
from jax.experimental import pallas as pl
from jax.experimental.pallas import tpu as pltpu
from jax.experimental.pallas import tpu_sc as plsc

B = 4096
L = 200
DIM = 32
ZERO_ROW = 512

NUM_CORES = 2
NUM_SUBCORES = 16
NW = NUM_CORES * NUM_SUBCORES
TOTAL = B * L
PER_W = TOTAL // NW
BLK = 512
NBLK = PER_W // BLK


def _sc_embed(xf, emb_table, pos_aug):
    mesh = plsc.VectorSubcoreMesh(core_axis_name="c", subcore_axis_name="s")

    @functools.partial(
        pl.kernel,
        mesh=mesh,
        out_type=jax.ShapeDtypeStruct((TOTAL, DIM), jnp.float32),
        compiler_params=pltpu.CompilerParams(use_tc_tiling_on_sc=False),
        scratch_types=[
            pltpu.VMEM((BLK,), jnp.int32),
            pltpu.VMEM((BLK,), jnp.int32),
            pltpu.VMEM((BLK, DIM), jnp.float32),
            pltpu.VMEM((BLK, DIM), jnp.float32),
            pltpu.SemaphoreType.DMA,
            pltpu.SemaphoreType.DMA,
        ],
    )
    def body(x_hbm, emb_hbm, pos_hbm, out_hbm, idx_v, pidx_v, rows_v, prow_v,
             sem_e, sem_p):
        wid = lax.axis_index("s") * NUM_CORES + lax.axis_index("c")
        base = wid * PER_W

        @pl.loop(0, NBLK)
        def _(blk):
            off = base + blk * BLK
            pltpu.sync_copy(x_hbm.at[pl.ds(off, BLK)], idx_v)
            cp_e = pltpu.async_copy(emb_hbm.at[idx_v], rows_v, sem_e)

            @pl.loop(0, BLK, step=16)
            def _(i):
                iv = idx_v[pl.ds(i, 16)]
                fpos = (off + i + lax.iota(jnp.int32, 16)) % L
                pidx_v[pl.ds(i, 16)] = jnp.where(
                    iv == jnp.int32(0), jnp.int32(ZERO_ROW), fpos)

            cp_p = pltpu.async_copy(pos_hbm.at[pidx_v], prow_v, sem_p)
            cp_e.wait()
            cp_p.wait()

            @pl.loop(0, BLK)
            def _(r):
                rows_v[r, pl.ds(0, 16)] += prow_v[r, pl.ds(0, 16)]
                rows_v[r, pl.ds(16, 16)] += prow_v[r, pl.ds(16, 16)]

            pltpu.sync_copy(rows_v, out_hbm.at[pl.ds(off, BLK)])

    return body(xf, emb_table, pos_aug)


def kernel(x, emb_table, pos_table):
    xf = x.reshape(TOTAL).astype(jnp.int32)
    pos_aug = jnp.concatenate(
        [pos_table, jnp.zeros((8, DIM), jnp.float32)], axis=0)
    out = _sc_embed(xf, emb_table, pos_aug)
    return out.reshape(B, L, DIM)


# speedup vs baseline: 1.1625x; 1.0153x over previous
import functools

import jax
import jax.numpy as jnp
from jax import lax
from jax.experimental import pallas as pl
from jax.experimental.pallas import tpu as pltpu
from jax.experimental.pallas import tpu_sc as plsc

B = 4096
L = 200
DIM = 32
ZERO_ROW = 512

NUM_CORES = 2
NUM_SUBCORES = 16
NW = NUM_CORES * NUM_SUBCORES
SEQ_PER_W = B // NW
SBLK = 4
ROWS = SBLK * L
NBLK = SEQ_PER_W // SBLK


def _sc_embed(xf, emb4m, pos_aug):
    mesh = plsc.VectorSubcoreMesh(core_axis_name="c", subcore_axis_name="s")

    @functools.partial(
        pl.kernel,
        mesh=mesh,
        out_type=jax.ShapeDtypeStruct((B, L, DIM), jnp.float32),
        compiler_params=pltpu.CompilerParams(use_tc_tiling_on_sc=False),
        scratch_types=[
            pltpu.VMEM((ROWS,), jnp.int32),
            pltpu.VMEM((ROWS,), jnp.int32),
            pltpu.VMEM((ROWS, DIM), jnp.float32),
            pltpu.VMEM((ROWS, DIM), jnp.float32),
            pltpu.SemaphoreType.DMA,
            pltpu.SemaphoreType.DMA,
        ],
    )
    def body(x_hbm, emb_hbm, pos_hbm, out_hbm, idx_v, pidx_v, rows_v, prow_v,
             sem_e, sem_p):
        wid = lax.axis_index("s") * NUM_CORES + lax.axis_index("c")
        seq0 = wid * SEQ_PER_W

        @pl.loop(0, NBLK)
        def _(blk):
            seq = seq0 + blk * SBLK
            off = seq * L
            pltpu.sync_copy(x_hbm.at[pl.ds(off, ROWS)], idx_v)

            @pl.loop(0, ROWS, step=16)
            def _(i):
                iv = idx_v[pl.ds(i, 16)]
                fpos = (i + lax.iota(jnp.int32, 16)) % L
                pidx_v[pl.ds(i, 16)] = jnp.where(
                    iv == jnp.int32(0), jnp.int32(ZERO_ROW), fpos)
                idx_v[pl.ds(i, 16)] = iv * jnp.int32(4)

            cp_e = pltpu.async_copy(emb_hbm.at[idx_v], rows_v, sem_e)
            cp_p = pltpu.async_copy(pos_hbm.at[pidx_v], prow_v, sem_p)
            cp_e.wait()
            cp_p.wait()

            @pl.loop(0, ROWS)
            def _(r):
                rows_v[r, pl.ds(0, 16)] += prow_v[r, pl.ds(0, 16)]
                rows_v[r, pl.ds(16, 16)] += prow_v[r, pl.ds(16, 16)]

            @pl.loop(0, SBLK)
            def _(s):
                pltpu.sync_copy(rows_v.at[pl.ds(s * L, L)],
                                out_hbm.at[seq + s])

    return body(xf, emb4m, pos_aug)


def kernel(x, emb_table, pos_table):
    xf = x.reshape(B * L).astype(jnp.int32)
    emb4m = jnp.pad(emb_table, ((0, 0), (0, 96))).reshape(4 * 1000000, DIM)
    pos_aug = jnp.concatenate(
        [pos_table, jnp.zeros((8, DIM), jnp.float32)], axis=0)
    return _sc_embed(xf, emb4m, pos_aug)
